# baseline (speedup 1.0000x reference)
import jax
import jax.numpy as jnp
from jax import lax
from jax.experimental import pallas as pl
from jax.experimental.pallas import tpu as pltpu
from jax.experimental.pallas import tpu_sc as plsc

_VOCAB = 1000000
_D = 64
_T = 128
_B = 4096
_S = 200

_NC = 2
_NS = 16
_NW = _NC * _NS
_BPW = _B // _NW


def _sc_pool_body(xw_hbm, table_hbm, out_hbm, idx_v, iota_v, acc_sh, r0, r1,
                  sem0, sem1):
    sid = lax.axis_index("s")
    wid = sid * _NC + lax.axis_index("c")
    base = wid * _BPW

    for k in range(_BPW // 16):
        iota_v[pl.ds(16 * k, 16)] = lax.iota(jnp.int32, 16) + (
            16 * k + sid * _BPW)

    pltpu.sync_copy(xw_hbm.at[wid], idx_v)

    pltpu.async_copy(table_hbm.at[idx_v.at[0]], r0, sem0).wait()
    pltpu.async_copy(table_hbm.at[idx_v.at[1]], r1, sem1)
    pltpu.sync_copy(r0, acc_sh.at[iota_v])

    @pl.loop(2, _S, step=2)
    def _(j):
        pltpu.async_copy(table_hbm.at[idx_v.at[j]], r0, sem0)
        pltpu.make_async_copy(table_hbm.at[idx_v.at[1]], r1, sem1).wait()
        pltpu.sync_copy(r1, acc_sh.at[iota_v], add=True)
        pltpu.async_copy(table_hbm.at[idx_v.at[j + 1]], r1, sem1)
        pltpu.make_async_copy(table_hbm.at[idx_v.at[0]], r0, sem0).wait()
        pltpu.sync_copy(r0, acc_sh.at[iota_v], add=True)

    pltpu.make_async_copy(table_hbm.at[idx_v.at[1]], r1, sem1).wait()
    pltpu.sync_copy(r1, acc_sh.at[iota_v], add=True)

    pltpu.sync_copy(acc_sh.at[pl.ds(sid * _BPW, _BPW)],
                    out_hbm.at[pl.ds(base, _BPW)])


def _sc_pool(xw, emb_table):
    mesh = plsc.VectorSubcoreMesh(core_axis_name="c", subcore_axis_name="s")
    return pl.kernel(
        _sc_pool_body,
        out_type=jax.ShapeDtypeStruct((_B, _D), jnp.float32),
        mesh=mesh,
        scratch_types=[
            pltpu.VMEM((_S, _BPW), jnp.int32),
            pltpu.VMEM((_BPW,), jnp.int32),
            pltpu.VMEM_SHARED((_NS * _BPW, _D), jnp.float32),
            pltpu.VMEM((_BPW, _D), jnp.float32),
            pltpu.VMEM((_BPW, _D), jnp.float32),
            pltpu.SemaphoreType.DMA,
            pltpu.SemaphoreType.DMA,
        ],
        compiler_params=pltpu.CompilerParams(use_tc_tiling_on_sc=False),
    )(xw, emb_table)


def _mm_body(p_ref, w_ref, b_ref, o_ref):
    o_ref[...] = (
        jnp.dot(p_ref[...], w_ref[...],
                preferred_element_type=jnp.float32,
                precision=lax.Precision.HIGHEST)
        + b_ref[...]
    )


def _tc_project(pooled, W, b):
    blk = 512
    return pl.pallas_call(
        _mm_body,
        out_shape=jax.ShapeDtypeStruct((_B, _T), jnp.float32),
        grid=(_B // blk,),
        in_specs=[
            pl.BlockSpec((blk, _D), lambda i: (i, 0)),
            pl.BlockSpec((_D, _T), lambda i: (0, 0)),
            pl.BlockSpec((1, _T), lambda i: (0, 0)),
        ],
        out_specs=pl.BlockSpec((blk, _T), lambda i: (i, 0)),
    )(pooled, W, b.reshape(1, _T))


def kernel(x, emb_table, W, b):
    xw = x.reshape(_NW, _BPW, _S).transpose(0, 2, 1)
    pooled = _sc_pool(xw, emb_table)
    return _tc_project(pooled, W, b)
